# Initial kernel scaffold; baseline (speedup 1.0000x reference)
import jax
import jax.numpy as jnp
from jax.experimental import pallas as pl


def kernel(ehr, ehr_len, img, w_ehr, w_cxr, w_proj, b_proj, w_cls, b_cls):
    raise NotImplementedError("write your pallas kernel here")


# speedup vs baseline: 1.0215x; 1.0215x over previous
import functools

import jax
import jax.numpy as jnp
from jax.experimental import pallas as pl
from jax.experimental.pallas import tpu as pltpu

LANE = 128
SUBLANE = 8


def _ru(x, m):
    return ((x + m - 1) // m) * m


def _fused_kernel(len_ref, ehr_ref, img_ref, w_ehr_ref, w_cxr_ref,
                  w_proj_ref, b_proj_ref, w_cls_ref, b_cls_ref,
                  preds_ref, proj_ref, ehrf_ref, *, T, FP, HW, HWP, C, DP):
    lens = len_ref[...]

    acc = jnp.zeros((ehr_ref.shape[0], FP), dtype=jnp.float32)
    for t in range(T):
        m_t = (lens > jnp.float32(t)).astype(jnp.float32)
        acc = acc + ehr_ref[:, t * FP:(t + 1) * FP] * m_t

    inv_len = 1.0 / jnp.maximum(lens, 1.0)
    ehr_feats = jnp.tanh(
        jnp.dot(acc, w_ehr_ref[...], preferred_element_type=jnp.float32) * inv_len)
    ehrf_ref[...] = ehr_feats

    inv_hw = jnp.float32(1.0) / jnp.float32(HW)
    cxr = jnp.zeros((img_ref.shape[0], w_cxr_ref.shape[1]), dtype=jnp.float32)
    for c in range(C):
        g = jnp.sum(img_ref[:, c * HWP:(c + 1) * HWP], axis=1,
                    keepdims=True) * inv_hw
        cxr = cxr + g * w_cxr_ref[c:c + 1, :]
    cxr = jax.nn.relu(cxr)

    proj = jnp.dot(cxr, w_proj_ref[...],
                   preferred_element_type=jnp.float32) + b_proj_ref[...]
    proj_ref[...] = proj

    logits = (jnp.dot(ehr_feats, w_cls_ref[0:DP, :], preferred_element_type=jnp.float32)
              + jnp.dot(proj, w_cls_ref[DP:2 * DP, :], preferred_element_type=jnp.float32)
              + b_cls_ref[...])
    preds_ref[...] = jax.nn.sigmoid(logits)


def kernel(ehr, ehr_len, img, w_ehr, w_cxr, w_proj, b_proj, w_cls, b_cls):
    B, T, F = ehr.shape
    C, H, W = img.shape[1], img.shape[2], img.shape[3]
    R, D = w_proj.shape
    C_out = b_cls.shape[1]
    HW = H * W

    FP = _ru(F, LANE)
    DP = _ru(D, LANE)
    RP = _ru(R, LANE)
    CP = _ru(C_out, LANE)
    HWP = _ru(HW, LANE)

    if (F, D) != (FP, DP):
        w_ehr = jnp.zeros((FP, DP), jnp.float32).at[:F, :D].set(w_ehr)
    w_cxr_p = jnp.zeros((SUBLANE, RP), jnp.float32).at[:C, :R].set(w_cxr)
    if (R, D) != (RP, DP):
        w_proj = jnp.zeros((RP, DP), jnp.float32).at[:R, :D].set(w_proj)
        b_proj = jnp.zeros((1, DP), jnp.float32).at[:, :D].set(b_proj)
    if (D, C_out) != (DP, CP):
        w_cls_p = (jnp.zeros((2 * DP, CP), jnp.float32)
                   .at[:D, :C_out].set(w_cls[:D])
                   .at[DP:DP + D, :C_out].set(w_cls[D:]))
        b_cls = jnp.zeros((1, CP), jnp.float32).at[:, :C_out].set(b_cls)
    else:
        w_cls_p = w_cls

    if FP != F:
        ehr = jnp.pad(ehr, ((0, 0), (0, 0), (0, FP - F)))
    ehr2 = ehr.reshape(B, T * FP)
    img3 = img.reshape(B, C, HW)
    if HWP != HW:
        img3 = jnp.pad(img3, ((0, 0), (0, 0), (0, HWP - HW)))
    img2 = img3.reshape(B, C * HWP)
    len2 = ehr_len.astype(jnp.float32).reshape(B, 1)

    TB = min(256, _ru(B, SUBLANE))
    BP = _ru(B, TB)
    if BP != B:
        ehr2 = jnp.pad(ehr2, ((0, BP - B), (0, 0)))
        img2 = jnp.pad(img2, ((0, BP - B), (0, 0)))
        len2 = jnp.pad(len2, ((0, BP - B), (0, 0)))

    grid = (BP // TB,)
    body = functools.partial(_fused_kernel, T=T, FP=FP, HW=HW, HWP=HWP, C=C, DP=DP)

    preds_p, proj_p, ehrf_p = pl.pallas_call(
        body,
        out_shape=(
            jax.ShapeDtypeStruct((BP, CP), jnp.float32),
            jax.ShapeDtypeStruct((BP, DP), jnp.float32),
            jax.ShapeDtypeStruct((BP, DP), jnp.float32),
        ),
        grid=grid,
        in_specs=[
            pl.BlockSpec((TB, 1), lambda i: (i, 0)),
            pl.BlockSpec((TB, T * FP), lambda i: (i, 0)),
            pl.BlockSpec((TB, C * HWP), lambda i: (i, 0)),
            pl.BlockSpec((FP, DP), lambda i: (0, 0)),
            pl.BlockSpec((SUBLANE, RP), lambda i: (0, 0)),
            pl.BlockSpec((RP, DP), lambda i: (0, 0)),
            pl.BlockSpec((1, DP), lambda i: (0, 0)),
            pl.BlockSpec((2 * DP, CP), lambda i: (0, 0)),
            pl.BlockSpec((1, CP), lambda i: (0, 0)),
        ],
        out_specs=[
            pl.BlockSpec((TB, CP), lambda i: (i, 0)),
            pl.BlockSpec((TB, DP), lambda i: (i, 0)),
            pl.BlockSpec((TB, DP), lambda i: (i, 0)),
        ],
        compiler_params=pltpu.CompilerParams(
            dimension_semantics=("parallel",)),
    )(len2, ehr2, img2, w_ehr, w_cxr_p, w_proj, b_proj, w_cls_p, b_cls)

    return {
        "preds": preds_p[:B, :C_out],
        "ehr_feats": ehrf_p[:B, :D],
        "cxr_feats": proj_p[:B, :D],
    }


# speedup vs baseline: 2.0614x; 2.0180x over previous
import functools

import jax
import jax.numpy as jnp
from jax.experimental import pallas as pl
from jax.experimental.pallas import tpu as pltpu

LANE = 128
SUBLANE = 8


def _ru(x, m):
    return ((x + m - 1) // m) * m


def _fused_kernel(len_ref, ehr_ref, img_ref, w_ehr_ref, w_cxr_ref,
                  w_proj_ref, b_proj_ref, w_cls_ref, b_cls_ref,
                  preds_ref, proj_ref, ehrf_ref, *, TP, FP, HW, HWP, C, DP):
    TB = len_ref.shape[0]
    lens = len_ref[...]

    ti = jax.lax.broadcasted_iota(jnp.int32, (TB, TP, 1), 1).astype(jnp.float32)
    mask3 = (lens[:, :, None] > ti).astype(jnp.float32)
    acc3 = jnp.zeros((TB, SUBLANE, FP), dtype=jnp.float32)
    for j in range(TP // SUBLANE):
        sl = pl.ds(j * SUBLANE, SUBLANE)
        acc3 = acc3 + ehr_ref[:, sl, :] * mask3[:, j * SUBLANE:(j + 1) * SUBLANE, :]
    acc = jnp.sum(acc3, axis=1)

    inv_len = 1.0 / jnp.maximum(lens, 1.0)
    ehr_feats = jnp.tanh(
        jnp.dot(acc, w_ehr_ref[...], preferred_element_type=jnp.float32) * inv_len)
    ehrf_ref[...] = ehr_feats

    inv_hw = jnp.float32(1.0 / HW)
    cxr = jnp.zeros((TB, w_cxr_ref.shape[1]), dtype=jnp.float32)
    for c in range(C):
        g = jnp.sum(img_ref[:, c * HWP:(c + 1) * HWP], axis=1,
                    keepdims=True) * inv_hw
        cxr = cxr + g * w_cxr_ref[c:c + 1, :]
    cxr = jax.nn.relu(cxr)

    proj = jnp.dot(cxr, w_proj_ref[...],
                   preferred_element_type=jnp.float32) + b_proj_ref[...]
    proj_ref[...] = proj

    logits = (jnp.dot(ehr_feats, w_cls_ref[0:DP, :], preferred_element_type=jnp.float32)
              + jnp.dot(proj, w_cls_ref[DP:2 * DP, :], preferred_element_type=jnp.float32)
              + b_cls_ref[...])
    preds_ref[...] = jax.nn.sigmoid(logits)


def kernel(ehr, ehr_len, img, w_ehr, w_cxr, w_proj, b_proj, w_cls, b_cls):
    B, T, F = ehr.shape
    C, H, W = img.shape[1], img.shape[2], img.shape[3]
    R, D = w_proj.shape
    C_out = b_cls.shape[1]
    HW = H * W

    FP = _ru(F, LANE)
    DP = _ru(D, LANE)
    RP = _ru(R, LANE)
    CP = _ru(C_out, LANE)
    HWP = _ru(HW, LANE)
    TP = _ru(T, SUBLANE)

    if (F, D) != (FP, DP):
        w_ehr = jnp.zeros((FP, DP), jnp.float32).at[:F, :D].set(w_ehr)
    w_cxr_p = jnp.zeros((SUBLANE, RP), jnp.float32).at[:C, :R].set(w_cxr)
    if (R, D) != (RP, DP):
        w_proj = jnp.zeros((RP, DP), jnp.float32).at[:R, :D].set(w_proj)
        b_proj = jnp.zeros((1, DP), jnp.float32).at[:, :D].set(b_proj)
    if (D, C_out) != (DP, CP):
        w_cls_p = (jnp.zeros((2 * DP, CP), jnp.float32)
                   .at[:D, :C_out].set(w_cls[:D])
                   .at[DP:DP + D, :C_out].set(w_cls[D:]))
        b_cls = jnp.zeros((1, CP), jnp.float32).at[:, :C_out].set(b_cls)
    else:
        w_cls_p = w_cls
    if (T, F) != (TP, FP):
        ehr = jnp.pad(ehr, ((0, 0), (0, TP - T), (0, FP - F)))

    img3 = img.reshape(B, C, HW)
    if HWP != HW:
        img3 = jnp.pad(img3, ((0, 0), (0, 0), (0, HWP - HW)))
    img2 = img3.reshape(B, C * HWP)

    len2 = ehr_len.astype(jnp.float32).reshape(B, 1)

    TB = min(128, _ru(B, SUBLANE))
    BP = _ru(B, TB)
    if BP != B:
        ehr = jnp.pad(ehr, ((0, BP - B), (0, 0), (0, 0)))
        img2 = jnp.pad(img2, ((0, BP - B), (0, 0)))
        len2 = jnp.pad(len2, ((0, BP - B), (0, 0)))

    grid = (BP // TB,)
    body = functools.partial(_fused_kernel, TP=TP, FP=FP, HW=HW, HWP=HWP,
                             C=C, DP=DP)

    preds_p, proj_p, ehrf_p = pl.pallas_call(
        body,
        out_shape=(
            jax.ShapeDtypeStruct((BP, CP), jnp.float32),
            jax.ShapeDtypeStruct((BP, DP), jnp.float32),
            jax.ShapeDtypeStruct((BP, DP), jnp.float32),
        ),
        grid=grid,
        in_specs=[
            pl.BlockSpec((TB, 1), lambda i: (i, 0)),
            pl.BlockSpec((TB, TP, FP), lambda i: (i, 0, 0)),
            pl.BlockSpec((TB, C * HWP), lambda i: (i, 0)),
            pl.BlockSpec((FP, DP), lambda i: (0, 0)),
            pl.BlockSpec((SUBLANE, RP), lambda i: (0, 0)),
            pl.BlockSpec((RP, DP), lambda i: (0, 0)),
            pl.BlockSpec((1, DP), lambda i: (0, 0)),
            pl.BlockSpec((2 * DP, CP), lambda i: (0, 0)),
            pl.BlockSpec((1, CP), lambda i: (0, 0)),
        ],
        out_specs=[
            pl.BlockSpec((TB, CP), lambda i: (i, 0)),
            pl.BlockSpec((TB, DP), lambda i: (i, 0)),
            pl.BlockSpec((TB, DP), lambda i: (i, 0)),
        ],
        compiler_params=pltpu.CompilerParams(
            dimension_semantics=("parallel",)),
    )(len2, ehr, img2, w_ehr, w_cxr_p, w_proj, b_proj, w_cls_p, b_cls)

    return {
        "preds": preds_p[:B, :C_out],
        "ehr_feats": ehrf_p[:B, :D],
        "cxr_feats": proj_p[:B, :D],
    }


# speedup vs baseline: 2.0915x; 1.0146x over previous
import functools

import jax
import jax.numpy as jnp
from jax.experimental import pallas as pl
from jax.experimental.pallas import tpu as pltpu

LANE = 128
SUBLANE = 8


def _ru(x, m):
    return ((x + m - 1) // m) * m


def _fused_kernel(len_ref, ehr_ref, img_ref, w_ehr_ref, w_cxr_ref,
                  w_proj_ref, b_proj_ref, w_cls_ref, b_cls_ref,
                  preds_ref, proj_ref, ehrf_ref, *, TP, FP, HW, HWP, C, DP):
    TB = len_ref.shape[0]
    lens = len_ref[...]

    ti = jax.lax.broadcasted_iota(jnp.int32, (TB, TP, 1), 1).astype(jnp.float32)
    mask3 = (lens[:, :, None] > ti).astype(jnp.float32)
    acc3 = jnp.zeros((TB, SUBLANE, FP), dtype=jnp.float32)
    for j in range(TP // SUBLANE):
        sl = pl.ds(j * SUBLANE, SUBLANE)
        acc3 = acc3 + ehr_ref[:, sl, :] * mask3[:, j * SUBLANE:(j + 1) * SUBLANE, :]
    acc = jnp.sum(acc3, axis=1)

    inv_len = 1.0 / jnp.maximum(lens, 1.0)
    ehr_feats = jnp.tanh(
        jnp.dot(acc, w_ehr_ref[...], preferred_element_type=jnp.float32) * inv_len)
    ehrf_ref[...] = ehr_feats

    inv_hw = jnp.float32(1.0 / HW)
    cxr = jnp.zeros((TB, w_cxr_ref.shape[1]), dtype=jnp.float32)
    for c in range(C):
        g = jnp.sum(img_ref[:, c * HWP:(c + 1) * HWP], axis=1,
                    keepdims=True) * inv_hw
        cxr = cxr + g * w_cxr_ref[c:c + 1, :]
    cxr = jax.nn.relu(cxr)

    proj = jnp.dot(cxr, w_proj_ref[...],
                   preferred_element_type=jnp.float32) + b_proj_ref[...]
    proj_ref[...] = proj

    logits = (jnp.dot(ehr_feats, w_cls_ref[0:DP, :], preferred_element_type=jnp.float32)
              + jnp.dot(proj, w_cls_ref[DP:2 * DP, :], preferred_element_type=jnp.float32)
              + b_cls_ref[...])
    preds_ref[...] = jax.nn.sigmoid(logits)


def kernel(ehr, ehr_len, img, w_ehr, w_cxr, w_proj, b_proj, w_cls, b_cls):
    B, T, F = ehr.shape
    C, H, W = img.shape[1], img.shape[2], img.shape[3]
    R, D = w_proj.shape
    C_out = b_cls.shape[1]
    HW = H * W

    FP = _ru(F, LANE)
    DP = _ru(D, LANE)
    RP = _ru(R, LANE)
    CP = _ru(C_out, LANE)
    HWP = _ru(HW, LANE)
    TP = _ru(T, SUBLANE)

    if (F, D) != (FP, DP):
        w_ehr = jnp.zeros((FP, DP), jnp.float32).at[:F, :D].set(w_ehr)
    w_cxr_p = jnp.zeros((SUBLANE, RP), jnp.float32).at[:C, :R].set(w_cxr)
    if (R, D) != (RP, DP):
        w_proj = jnp.zeros((RP, DP), jnp.float32).at[:R, :D].set(w_proj)
        b_proj = jnp.zeros((1, DP), jnp.float32).at[:, :D].set(b_proj)
    if (D, C_out) != (DP, CP):
        w_cls_p = (jnp.zeros((2 * DP, CP), jnp.float32)
                   .at[:D, :C_out].set(w_cls[:D])
                   .at[DP:DP + D, :C_out].set(w_cls[D:]))
        b_cls = jnp.zeros((1, CP), jnp.float32).at[:, :C_out].set(b_cls)
    else:
        w_cls_p = w_cls
    if (T, F) != (TP, FP):
        ehr = jnp.pad(ehr, ((0, 0), (0, TP - T), (0, FP - F)))

    img3 = img.reshape(B, C, HW)
    if HWP != HW:
        img3 = jnp.pad(img3, ((0, 0), (0, 0), (0, HWP - HW)))
    img2 = img3.reshape(B, C * HWP)

    len2 = ehr_len.astype(jnp.float32).reshape(B, 1)

    TB = min(256, _ru(B, SUBLANE))
    BP = _ru(B, TB)
    if BP != B:
        ehr = jnp.pad(ehr, ((0, BP - B), (0, 0), (0, 0)))
        img2 = jnp.pad(img2, ((0, BP - B), (0, 0)))
        len2 = jnp.pad(len2, ((0, BP - B), (0, 0)))

    grid = (BP // TB,)
    body = functools.partial(_fused_kernel, TP=TP, FP=FP, HW=HW, HWP=HWP,
                             C=C, DP=DP)

    preds_p, proj_p, ehrf_p = pl.pallas_call(
        body,
        out_shape=(
            jax.ShapeDtypeStruct((BP, CP), jnp.float32),
            jax.ShapeDtypeStruct((BP, DP), jnp.float32),
            jax.ShapeDtypeStruct((BP, DP), jnp.float32),
        ),
        grid=grid,
        in_specs=[
            pl.BlockSpec((TB, 1), lambda i: (i, 0)),
            pl.BlockSpec((TB, TP, FP), lambda i: (i, 0, 0)),
            pl.BlockSpec((TB, C * HWP), lambda i: (i, 0)),
            pl.BlockSpec((FP, DP), lambda i: (0, 0)),
            pl.BlockSpec((SUBLANE, RP), lambda i: (0, 0)),
            pl.BlockSpec((RP, DP), lambda i: (0, 0)),
            pl.BlockSpec((1, DP), lambda i: (0, 0)),
            pl.BlockSpec((2 * DP, CP), lambda i: (0, 0)),
            pl.BlockSpec((1, CP), lambda i: (0, 0)),
        ],
        out_specs=[
            pl.BlockSpec((TB, CP), lambda i: (i, 0)),
            pl.BlockSpec((TB, DP), lambda i: (i, 0)),
            pl.BlockSpec((TB, DP), lambda i: (i, 0)),
        ],
        compiler_params=pltpu.CompilerParams(
            dimension_semantics=("parallel",)),
    )(len2, ehr, img2, w_ehr, w_cxr_p, w_proj, b_proj, w_cls_p, b_cls)

    return {
        "preds": preds_p[:B, :C_out],
        "ehr_feats": ehrf_p[:B, :D],
        "cxr_feats": proj_p[:B, :D],
    }


# speedup vs baseline: 2.9438x; 1.4075x over previous
import functools

import jax
import jax.numpy as jnp
from jax.experimental import pallas as pl
from jax.experimental.pallas import tpu as pltpu

LANE = 128
SUBLANE = 8


def _ru(x, m):
    return ((x + m - 1) // m) * m


def _fused_kernel(len_ref, ehr_ref, imgT_ref, w_ehr_ref, w_cxr_t_ref,
                  w_proj_t_ref, b_proj_t_ref, w_cls_ref, b_cls_ref,
                  preds_ref, proj_ref, ehrf_ref, *, TP, FP, HW, HWP, C, DP):
    TB = len_ref.shape[0]
    lens = len_ref[...]

    ti = jax.lax.broadcasted_iota(jnp.int32, (TB, TP, 1), 1).astype(jnp.float32)
    mask3 = (lens[:, :, None] > ti).astype(jnp.float32)
    acc3 = jnp.zeros((TB, SUBLANE, FP), dtype=jnp.float32)
    for j in range(TP // SUBLANE):
        sl = pl.ds(j * SUBLANE, SUBLANE)
        acc3 = acc3 + ehr_ref[:, sl, :] * mask3[:, j * SUBLANE:(j + 1) * SUBLANE, :]
    acc = jnp.sum(acc3, axis=1)

    inv_len = 1.0 / jnp.maximum(lens, 1.0)
    ehr_feats = jnp.tanh(
        jnp.dot(acc, w_ehr_ref[...], preferred_element_type=jnp.float32) * inv_len)
    ehrf_ref[...] = ehr_feats

    inv_hw = jnp.float32(1.0 / HW)
    gparts = [jnp.sum(imgT_ref[c * HWP:(c + 1) * HWP, :], axis=0, keepdims=True)
              for c in range(C)]
    gparts.append(jnp.zeros((SUBLANE - C, TB), jnp.float32))
    g8 = jnp.concatenate(gparts, axis=0) * inv_hw
    cxrT = jax.nn.relu(jnp.dot(w_cxr_t_ref[...], g8,
                               preferred_element_type=jnp.float32))
    projT = jnp.dot(w_proj_t_ref[...], cxrT,
                    preferred_element_type=jnp.float32) + b_proj_t_ref[...]

    proj_ref[...] = jnp.transpose(projT)

    logits = (jnp.dot(ehr_feats, w_cls_ref[0:DP, :], preferred_element_type=jnp.float32)
              + jax.lax.dot_general(projT, w_cls_ref[DP:2 * DP, :],
                                    dimension_numbers=(((0,), (0,)), ((), ())),
                                    preferred_element_type=jnp.float32)
              + b_cls_ref[...])
    preds_ref[...] = jax.nn.sigmoid(logits)


def kernel(ehr, ehr_len, img, w_ehr, w_cxr, w_proj, b_proj, w_cls, b_cls):
    B, T, F = ehr.shape
    C, H, W = img.shape[1], img.shape[2], img.shape[3]
    R, D = w_proj.shape
    C_out = b_cls.shape[1]
    HW = H * W

    FP = _ru(F, LANE)
    DP = _ru(D, LANE)
    RP = _ru(R, LANE)
    CP = _ru(C_out, LANE)
    HWP = _ru(HW, LANE)
    TP = _ru(T, SUBLANE)

    if (F, D) != (FP, DP):
        w_ehr = jnp.zeros((FP, DP), jnp.float32).at[:F, :D].set(w_ehr)
    w_cxr_t = jnp.zeros((RP, SUBLANE), jnp.float32).at[:R, :C].set(w_cxr.T)
    w_proj_t = jnp.zeros((DP, RP), jnp.float32).at[:D, :R].set(w_proj.T)
    b_proj_t = jnp.zeros((DP, 1), jnp.float32).at[:D, :].set(b_proj.T)
    if (D, C_out) != (DP, CP):
        w_cls_p = (jnp.zeros((2 * DP, CP), jnp.float32)
                   .at[:D, :C_out].set(w_cls[:D])
                   .at[DP:DP + D, :C_out].set(w_cls[D:]))
        b_cls = jnp.zeros((1, CP), jnp.float32).at[:, :C_out].set(b_cls)
    else:
        w_cls_p = w_cls
    if (T, F) != (TP, FP):
        ehr = jnp.pad(ehr, ((0, 0), (0, TP - T), (0, FP - F)))

    if HWP != HW:
        img3 = jnp.pad(img.reshape(B, C, HW), ((0, 0), (0, 0), (0, HWP - HW)))
        imgT = img3.reshape(B, C * HWP).T
    else:
        imgT = img.reshape(B, C * HW).T

    len2 = ehr_len.astype(jnp.float32).reshape(B, 1)

    TB = min(256, _ru(B, SUBLANE))
    BP = _ru(B, TB)
    if BP != B:
        ehr = jnp.pad(ehr, ((0, BP - B), (0, 0), (0, 0)))
        imgT = jnp.pad(imgT, ((0, 0), (0, BP - B)))
        len2 = jnp.pad(len2, ((0, BP - B), (0, 0)))

    grid = (BP // TB,)
    body = functools.partial(_fused_kernel, TP=TP, FP=FP, HW=HW, HWP=HWP,
                             C=C, DP=DP)

    preds_p, proj_p, ehrf_p = pl.pallas_call(
        body,
        out_shape=(
            jax.ShapeDtypeStruct((BP, CP), jnp.float32),
            jax.ShapeDtypeStruct((BP, DP), jnp.float32),
            jax.ShapeDtypeStruct((BP, DP), jnp.float32),
        ),
        grid=grid,
        in_specs=[
            pl.BlockSpec((TB, 1), lambda i: (i, 0)),
            pl.BlockSpec((TB, TP, FP), lambda i: (i, 0, 0)),
            pl.BlockSpec((C * HWP, TB), lambda i: (0, i)),
            pl.BlockSpec((FP, DP), lambda i: (0, 0)),
            pl.BlockSpec((RP, SUBLANE), lambda i: (0, 0)),
            pl.BlockSpec((DP, RP), lambda i: (0, 0)),
            pl.BlockSpec((DP, 1), lambda i: (0, 0)),
            pl.BlockSpec((2 * DP, CP), lambda i: (0, 0)),
            pl.BlockSpec((1, CP), lambda i: (0, 0)),
        ],
        out_specs=[
            pl.BlockSpec((TB, CP), lambda i: (i, 0)),
            pl.BlockSpec((TB, DP), lambda i: (i, 0)),
            pl.BlockSpec((TB, DP), lambda i: (i, 0)),
        ],
        compiler_params=pltpu.CompilerParams(
            dimension_semantics=("parallel",)),
    )(len2, ehr, imgT, w_ehr, w_cxr_t, w_proj_t, b_proj_t, w_cls_p, b_cls)

    return {
        "preds": preds_p[:B, :C_out],
        "ehr_feats": ehrf_p[:B, :D],
        "cxr_feats": proj_p[:B, :D],
    }


# speedup vs baseline: 3.1628x; 1.0744x over previous
import functools

import jax
import jax.numpy as jnp
from jax.experimental import pallas as pl
from jax.experimental.pallas import tpu as pltpu

LANE = 128
SUBLANE = 8


def _ru(x, m):
    return ((x + m - 1) // m) * m


def _fused_kernel(len_ref, ehr_ref, imgT_ref, w_ehr_ref, w_cxr_ref,
                  w_proj_ref, b_proj_ref, w_cls_ref, b_cls_ref,
                  preds_ref, proj_ref, ehrf_ref, *, TP, FP, HW, HWP, C, DP):
    TB = len_ref.shape[0]
    lens = len_ref[...]

    ti = jax.lax.broadcasted_iota(jnp.int32, (TB, TP, 1), 1)
    mask3 = (lens[:, :, None] > ti).astype(jnp.float32)
    acc3 = jnp.zeros((TB, SUBLANE, FP), dtype=jnp.float32)
    for j in range(TP // SUBLANE):
        sl = pl.ds(j * SUBLANE, SUBLANE)
        acc3 = acc3 + ehr_ref[:, sl, :] * mask3[:, j * SUBLANE:(j + 1) * SUBLANE, :]
    acc = jnp.sum(acc3, axis=1)

    inv_len = 1.0 / jnp.maximum(lens.astype(jnp.float32), 1.0)
    ehr_feats = jnp.tanh(
        jnp.dot(acc, w_ehr_ref[...], preferred_element_type=jnp.float32) * inv_len)
    ehrf_ref[...] = ehr_feats

    inv_hw = jnp.float32(1.0 / HW)
    gparts = [jnp.sum(imgT_ref[c * HWP:(c + 1) * HWP, :], axis=0, keepdims=True)
              for c in range(C)]
    g3 = jnp.concatenate(gparts, axis=0) * inv_hw
    cxrT = jax.nn.relu(
        jax.lax.dot_general(w_cxr_ref[...], g3,
                            dimension_numbers=(((0,), (0,)), ((), ())),
                            preferred_element_type=jnp.float32))
    projT = (jax.lax.dot_general(w_proj_ref[...], cxrT,
                                 dimension_numbers=(((0,), (0,)), ((), ())),
                                 preferred_element_type=jnp.float32)
             + jnp.transpose(b_proj_ref[...]))

    proj_ref[...] = jnp.transpose(projT)

    logits = (jnp.dot(ehr_feats, w_cls_ref[0:DP, :], preferred_element_type=jnp.float32)
              + jax.lax.dot_general(projT, w_cls_ref[DP:2 * DP, :],
                                    dimension_numbers=(((0,), (0,)), ((), ())),
                                    preferred_element_type=jnp.float32)
              + b_cls_ref[...])
    preds_ref[...] = jax.nn.sigmoid(logits)


def kernel(ehr, ehr_len, img, w_ehr, w_cxr, w_proj, b_proj, w_cls, b_cls):
    B, T, F = ehr.shape
    C, H, W = img.shape[1], img.shape[2], img.shape[3]
    R, D = w_proj.shape
    C_out = b_cls.shape[1]
    HW = H * W

    FP = _ru(F, LANE)
    DP = _ru(D, LANE)
    RP = _ru(R, LANE)
    CP = _ru(C_out, LANE)
    HWP = _ru(HW, LANE)
    TP = _ru(T, SUBLANE)

    if (F, D) != (FP, DP):
        w_ehr = jnp.zeros((FP, DP), jnp.float32).at[:F, :D].set(w_ehr)
    if (R, D) != (RP, DP):
        w_proj = jnp.zeros((RP, DP), jnp.float32).at[:R, :D].set(w_proj)
        b_proj = jnp.zeros((1, DP), jnp.float32).at[:, :D].set(b_proj)
        w_cxr = jnp.zeros((C, RP), jnp.float32).at[:, :R].set(w_cxr)
    if (D, C_out) != (DP, CP):
        w_cls_p = (jnp.zeros((2 * DP, CP), jnp.float32)
                   .at[:D, :C_out].set(w_cls[:D])
                   .at[DP:DP + D, :C_out].set(w_cls[D:]))
        b_cls = jnp.zeros((1, CP), jnp.float32).at[:, :C_out].set(b_cls)
    else:
        w_cls_p = w_cls
    if (T, F) != (TP, FP):
        ehr = jnp.pad(ehr, ((0, 0), (0, TP - T), (0, FP - F)))

    if HWP != HW:
        img3 = jnp.pad(img.reshape(B, C, HW), ((0, 0), (0, 0), (0, HWP - HW)))
        imgT = img3.reshape(B, C * HWP).T
    else:
        imgT = img.reshape(B, C * HW).T

    len2 = ehr_len.astype(jnp.int32).reshape(B, 1)

    TB = min(256, _ru(B, SUBLANE))
    BP = _ru(B, TB)
    if BP != B:
        ehr = jnp.pad(ehr, ((0, BP - B), (0, 0), (0, 0)))
        imgT = jnp.pad(imgT, ((0, 0), (0, BP - B)))
        len2 = jnp.pad(len2, ((0, BP - B), (0, 0)))

    grid = (BP // TB,)
    body = functools.partial(_fused_kernel, TP=TP, FP=FP, HW=HW, HWP=HWP,
                             C=C, DP=DP)

    preds_p, proj_p, ehrf_p = pl.pallas_call(
        body,
        out_shape=(
            jax.ShapeDtypeStruct((BP, CP), jnp.float32),
            jax.ShapeDtypeStruct((BP, DP), jnp.float32),
            jax.ShapeDtypeStruct((BP, DP), jnp.float32),
        ),
        grid=grid,
        in_specs=[
            pl.BlockSpec((TB, 1), lambda i: (i, 0)),
            pl.BlockSpec((TB, TP, FP), lambda i: (i, 0, 0)),
            pl.BlockSpec((C * HWP, TB), lambda i: (0, i)),
            pl.BlockSpec((FP, DP), lambda i: (0, 0)),
            pl.BlockSpec((C, RP), lambda i: (0, 0)),
            pl.BlockSpec((RP, DP), lambda i: (0, 0)),
            pl.BlockSpec((1, DP), lambda i: (0, 0)),
            pl.BlockSpec((2 * DP, CP), lambda i: (0, 0)),
            pl.BlockSpec((1, CP), lambda i: (0, 0)),
        ],
        out_specs=[
            pl.BlockSpec((TB, CP), lambda i: (i, 0)),
            pl.BlockSpec((TB, DP), lambda i: (i, 0)),
            pl.BlockSpec((TB, DP), lambda i: (i, 0)),
        ],
        compiler_params=pltpu.CompilerParams(
            dimension_semantics=("parallel",)),
    )(len2, ehr, imgT, w_ehr, w_cxr, w_proj, b_proj, w_cls_p, b_cls)

    return {
        "preds": preds_p[:B, :C_out],
        "ehr_feats": ehrf_p[:B, :D],
        "cxr_feats": proj_p[:B, :D],
    }
